# baseline (speedup 1.0000x reference)
import functools

import jax
import jax.numpy as jnp
from jax import lax
from jax.experimental import pallas as pl
from jax.experimental.pallas import tpu as pltpu
from jax.experimental.pallas import tpu_sc as plsc

B = 16
MAXN = 625
F = 128
H = 128
T = 8
E = 320000
L = 2
N = B * MAXN
PHID = 64

NWORK = 32
CHUNK = 80
NCHUNK = 128
NBUF = 4
E_PAD = NWORK * NCHUNK * CHUNK
N_PAD = 10112
RPS = N_PAD // 16
SEG = 16
NSEG = NCHUNK // SEG
RB = 2000
NB = N // RB


def _encode_body(x_ref, w_ref, b_ref, o_ref):
    o_ref[...] = jnp.tanh(
        jnp.dot(x_ref[...], w_ref[...], preferred_element_type=jnp.float32)
        + b_ref[...])


def _encode(x, w, b):
    return pl.pallas_call(
        _encode_body,
        grid=(NB,),
        in_specs=[pl.BlockSpec((RB, F), lambda i: (i, 0)),
                  pl.BlockSpec((F, H), lambda i: (0, 0)),
                  pl.BlockSpec((1, H), lambda i: (0, 0))],
        out_specs=pl.BlockSpec((RB, H), lambda i: (i, 0)),
        out_shape=jax.ShapeDtypeStruct((N, H), jnp.float32),
    )(x, w, b.reshape(1, H))


def _msgs_body(h_ref, w_ref, b_ref, o_ref):
    o_ref[...] = (
        jnp.dot(h_ref[...], w_ref[0], preferred_element_type=jnp.float32)
        + b_ref[0])


def _msgs(h, w, b):
    return pl.pallas_call(
        _msgs_body,
        grid=(T, NB),
        in_specs=[pl.BlockSpec((RB, H), lambda t, i: (i, 0)),
                  pl.BlockSpec((1, H, H), lambda t, i: (t, 0, 0)),
                  pl.BlockSpec((1, 1, H), lambda t, i: (t, 0, 0))],
        out_specs=pl.BlockSpec((RB, H), lambda t, i: (t * NB + i, 0)),
        out_shape=jax.ShapeDtypeStruct((T * N, H), jnp.float32),
    )(h, w, b.reshape(T, 1, H))


def _gru_body(p_ref, h_ref, wz_ref, uz_ref, bz_ref, wr_ref, ur_ref, br_ref,
              wn_ref, un_ref, bn_ref, o_ref):
    agg = p_ref[0] + p_ref[1]
    h = h_ref[...]
    dot = functools.partial(jnp.dot, preferred_element_type=jnp.float32)
    z = jax.nn.sigmoid(dot(agg, wz_ref[...]) + dot(h, uz_ref[...]) + bz_ref[...])
    r = jax.nn.sigmoid(dot(agg, wr_ref[...]) + dot(h, ur_ref[...]) + br_ref[...])
    c = jnp.tanh(dot(agg, wn_ref[...]) + r * dot(h, un_ref[...]) + bn_ref[...])
    o_ref[...] = (1.0 - z) * c + z * h


def _gru(partials, h, wz, uz, bz, wr, ur, br, wn, un, bn):
    wspec = pl.BlockSpec((H, H), lambda i: (0, 0))
    bspec = pl.BlockSpec((1, H), lambda i: (0, 0))
    return pl.pallas_call(
        _gru_body,
        grid=(NB,),
        in_specs=[pl.BlockSpec((2, RB, H), lambda i: (0, i, 0)),
                  pl.BlockSpec((RB, H), lambda i: (i, 0)),
                  wspec, wspec, bspec, wspec, wspec, bspec,
                  wspec, wspec, bspec],
        out_specs=pl.BlockSpec((RB, H), lambda i: (i, 0)),
        out_shape=jax.ShapeDtypeStruct((N, H), jnp.float32),
    )(partials, h, wz, uz, bz.reshape(1, H), wr, ur, br.reshape(1, H),
      wn, un, bn.reshape(1, H))


def _head_body(hb_ref, nn_ref, inh_ref, wp1_ref, bp1_ref, wp2_ref, bp2_ref,
               wprob_ref, bprob_ref, wconf_ref, bconf_ref, wc1_ref, bc1_ref,
               wc2_ref, bc2_ref, logits_ref, prob_ref, conf_ref, uu_ref,
               lower_ref):
    hb = hb_ref[...]
    nn = nn_ref[...]
    cols = lax.broadcasted_iota(jnp.int32, (B, MAXN), 1)
    mask = (cols < nn).astype(jnp.float32)
    pooled = jnp.sum(hb * mask[:, :, None], axis=1)
    denom = jnp.maximum(jnp.sum(mask, axis=1, keepdims=True), 1.0)
    lower = pooled / denom
    dot = functools.partial(jnp.dot, preferred_element_type=jnp.float32)
    pin = jnp.concatenate([inh_ref[...], lower], axis=1)
    hp = jax.nn.relu(dot(pin, wp1_ref[...]) + bp1_ref[...])
    hp = jax.nn.relu(dot(hp, wp2_ref[...]) + bp2_ref[...])
    prob = jax.nn.sigmoid(dot(hp, wprob_ref[...]) + bprob_ref[...])
    conf = jax.nn.sigmoid(dot(hp, wconf_ref[...]) + bconf_ref[...])
    hc = jax.nn.relu(dot(lower, wc1_ref[...]) + bc1_ref[...])
    logits_ref[...] = jax.nn.sigmoid(dot(hc, wc2_ref[...]) + bc2_ref[...])
    prob_ref[...] = prob
    conf_ref[...] = conf
    uu_ref[...] = (prob >= 0.5).astype(jnp.float32)
    lower_ref[...] = lower


def _head(hb, num_nodes, inh, wp1, bp1, wp2, bp2, wprob, bprob, wconf, bconf,
          wc1, bc1, wc2, bc2):
    full = lambda s: pl.BlockSpec(s, lambda: tuple(0 for _ in s))
    out_shapes = (jax.ShapeDtypeStruct((B, 1), jnp.float32),
                  jax.ShapeDtypeStruct((B, 1), jnp.float32),
                  jax.ShapeDtypeStruct((B, 1), jnp.float32),
                  jax.ShapeDtypeStruct((B, 1), jnp.float32),
                  jax.ShapeDtypeStruct((B, H), jnp.float32))
    return pl.pallas_call(
        _head_body,
        in_specs=[full((B, MAXN, H)), full((B, 1)), full((B, 8)),
                  full((8 + H, PHID)), full((1, PHID)),
                  full((PHID, PHID)), full((1, PHID)),
                  full((PHID, 1)), full((1, 1)),
                  full((PHID, 1)), full((1, 1)),
                  full((H, H // 2)), full((1, H // 2)),
                  full((H // 2, 1)), full((1, 1))],
        out_specs=(full((B, 1)), full((B, 1)), full((B, 1)), full((B, 1)),
                   full((B, H))),
        out_shape=out_shapes,
    )(hb, num_nodes.reshape(B, 1), inh, wp1, bp1.reshape(1, PHID),
      wp2, bp2.reshape(1, PHID), wprob, bprob.reshape(1, 1),
      wconf, bconf.reshape(1, 1), wc1, bc1.reshape(1, H // 2),
      wc2, bc2.reshape(1, 1))


def _sc_body(ht_hbm, flat_hbm, dst_hbm, zeros_hbm, out_hbm,
             idx_v, dst_v, rows_v, acc_sh, sem):
    c = lax.axis_index("c")
    s = lax.axis_index("s")
    wid = s * 2 + c
    pltpu.sync_copy(zeros_hbm, acc_sh.at[pl.ds(s * RPS, RPS)])
    plsc.subcore_barrier()
    for seg in range(NSEG):
        pltpu.sync_copy(flat_hbm.at[wid, pl.ds(seg * SEG, SEG)], idx_v)
        pltpu.sync_copy(dst_hbm.at[wid, pl.ds(seg * SEG, SEG)], dst_v)
        for b in range(NBUF):
            pltpu.async_copy(ht_hbm.at[idx_v.at[b]], rows_v.at[b], sem)

        def outer(j0, carry):
            for b in range(NBUF):
                j = j0 * NBUF + b
                pltpu.make_async_copy(ht_hbm.at[idx_v.at[j]], rows_v.at[b],
                                      sem).wait()
                pltpu.sync_copy(rows_v.at[b], acc_sh.at[dst_v.at[j]],
                                add=True)

                @pl.when(j + NBUF < SEG)
                def _():
                    pltpu.async_copy(ht_hbm.at[idx_v.at[j + NBUF]],
                                     rows_v.at[b], sem)
            return carry

        lax.fori_loop(0, SEG // NBUF, outer, 0)
    plsc.subcore_barrier()
    pltpu.sync_copy(acc_sh.at[pl.ds(s * RPS, RPS)],
                    out_hbm.at[c, pl.ds(s * RPS, RPS)])


def _sc_aggregate(ht, flat3, dst3, zeros):
    mesh = plsc.VectorSubcoreMesh(core_axis_name="c", subcore_axis_name="s")
    kern = pl.kernel(
        _sc_body,
        out_type=jax.ShapeDtypeStruct((2, N_PAD, H), jnp.float32),
        mesh=mesh,
        scratch_types=[
            pltpu.VMEM((SEG, CHUNK), jnp.int32),
            pltpu.VMEM((SEG, CHUNK), jnp.int32),
            pltpu.VMEM((NBUF, CHUNK, H), jnp.float32),
            pltpu.VMEM_SHARED((N_PAD, H), jnp.float32),
            pltpu.SemaphoreType.DMA,
        ],
    )
    return kern(ht, flat3, dst3, zeros)


def kernel(node_features, edge_index, edge_type, num_nodes,
           inheritance_features, W_in, b_in, W_msg, b_msg, Wz, Uz, bz,
           Wr, Ur, br, Wn, Un, bn, Wp1, bp1, Wp2, bp2, Wprob, bprob,
           Wconf, bconf, Wc1, bc1, Wc2, bc2):
    x = node_features.reshape(N, F)
    src = edge_index[0]
    dst = edge_index[1]
    flat = edge_type * N + src
    epw = E // NWORK
    ppw = (E_PAD - E) // NWORK
    pad_flat = jnp.zeros((NWORK, ppw), jnp.int32)
    pad_dst = jnp.broadcast_to(
        N + (jnp.arange(ppw, dtype=jnp.int32) % (N_PAD - N)), (NWORK, ppw))
    flat3 = jnp.concatenate(
        [flat.reshape(NWORK, epw), pad_flat], axis=1).reshape(
            NWORK, NCHUNK, CHUNK)
    dst3 = jnp.concatenate(
        [dst.reshape(NWORK, epw), pad_dst], axis=1).reshape(
            NWORK, NCHUNK, CHUNK)
    zeros = jnp.zeros((RPS, H), jnp.float32)

    h = _encode(x, W_in, b_in)
    for l in range(L):
        ht = _msgs(h, W_msg[l], b_msg[l])
        partials = _sc_aggregate(ht, flat3, dst3, zeros)
        h = _gru(partials, h, Wz[l], Uz[l], bz[l], Wr[l], Ur[l], br[l],
                 Wn[l], Un[l], bn[l])

    logits, prob, conf, uu, lower = _head(
        h.reshape(B, MAXN, H), num_nodes, inheritance_features,
        Wp1, bp1, Wp2, bp2, Wprob, bprob, Wconf, bconf, Wc1, bc1, Wc2, bc2)
    return logits, prob, conf, uu, lower


# speedup vs baseline: 1.0090x; 1.0090x over previous
import functools

import jax
import jax.numpy as jnp
from jax import lax
from jax.experimental import pallas as pl
from jax.experimental.pallas import tpu as pltpu
from jax.experimental.pallas import tpu_sc as plsc

B = 16
MAXN = 625
F = 128
H = 128
T = 8
E = 320000
L = 2
N = B * MAXN
PHID = 64

NWORK = 32
CHUNK = 128
NCHUNK = 80
NBUF = 2
E_PAD = NWORK * NCHUNK * CHUNK
N_PAD = 10112
RPS = N_PAD // 16
SEG = 16
NSEG = NCHUNK // SEG
RB = 2000
NB = N // RB


def _encode_body(x_ref, w_ref, b_ref, o_ref):
    o_ref[...] = jnp.tanh(
        jnp.dot(x_ref[...], w_ref[...], preferred_element_type=jnp.float32)
        + b_ref[...])


def _encode(x, w, b):
    return pl.pallas_call(
        _encode_body,
        grid=(NB,),
        in_specs=[pl.BlockSpec((RB, F), lambda i: (i, 0)),
                  pl.BlockSpec((F, H), lambda i: (0, 0)),
                  pl.BlockSpec((1, H), lambda i: (0, 0))],
        out_specs=pl.BlockSpec((RB, H), lambda i: (i, 0)),
        out_shape=jax.ShapeDtypeStruct((N, H), jnp.float32),
    )(x, w, b.reshape(1, H))


def _msgs_body(h_ref, w_ref, b_ref, o_ref):
    o_ref[...] = (
        jnp.dot(h_ref[...], w_ref[0], preferred_element_type=jnp.float32)
        + b_ref[0])


def _msgs(h, w, b):
    return pl.pallas_call(
        _msgs_body,
        grid=(T, NB),
        in_specs=[pl.BlockSpec((RB, H), lambda t, i: (i, 0)),
                  pl.BlockSpec((1, H, H), lambda t, i: (t, 0, 0)),
                  pl.BlockSpec((1, 1, H), lambda t, i: (t, 0, 0))],
        out_specs=pl.BlockSpec((RB, H), lambda t, i: (t * NB + i, 0)),
        out_shape=jax.ShapeDtypeStruct((T * N, H), jnp.float32),
    )(h, w, b.reshape(T, 1, H))


def _gru_body(p_ref, h_ref, wz_ref, uz_ref, bz_ref, wr_ref, ur_ref, br_ref,
              wn_ref, un_ref, bn_ref, o_ref):
    agg = p_ref[0] + p_ref[1]
    h = h_ref[...]
    dot = functools.partial(jnp.dot, preferred_element_type=jnp.float32)
    z = jax.nn.sigmoid(dot(agg, wz_ref[...]) + dot(h, uz_ref[...]) + bz_ref[...])
    r = jax.nn.sigmoid(dot(agg, wr_ref[...]) + dot(h, ur_ref[...]) + br_ref[...])
    c = jnp.tanh(dot(agg, wn_ref[...]) + r * dot(h, un_ref[...]) + bn_ref[...])
    o_ref[...] = (1.0 - z) * c + z * h


def _gru(partials, h, wz, uz, bz, wr, ur, br, wn, un, bn):
    wspec = pl.BlockSpec((H, H), lambda i: (0, 0))
    bspec = pl.BlockSpec((1, H), lambda i: (0, 0))
    return pl.pallas_call(
        _gru_body,
        grid=(NB,),
        in_specs=[pl.BlockSpec((2, RB, H), lambda i: (0, i, 0)),
                  pl.BlockSpec((RB, H), lambda i: (i, 0)),
                  wspec, wspec, bspec, wspec, wspec, bspec,
                  wspec, wspec, bspec],
        out_specs=pl.BlockSpec((RB, H), lambda i: (i, 0)),
        out_shape=jax.ShapeDtypeStruct((N, H), jnp.float32),
    )(partials, h, wz, uz, bz.reshape(1, H), wr, ur, br.reshape(1, H),
      wn, un, bn.reshape(1, H))


def _head_body(hb_ref, nn_ref, inh_ref, wp1_ref, bp1_ref, wp2_ref, bp2_ref,
               wprob_ref, bprob_ref, wconf_ref, bconf_ref, wc1_ref, bc1_ref,
               wc2_ref, bc2_ref, logits_ref, prob_ref, conf_ref, uu_ref,
               lower_ref):
    hb = hb_ref[...]
    nn = nn_ref[...]
    cols = lax.broadcasted_iota(jnp.int32, (B, MAXN), 1)
    mask = (cols < nn).astype(jnp.float32)
    pooled = jnp.sum(hb * mask[:, :, None], axis=1)
    denom = jnp.maximum(jnp.sum(mask, axis=1, keepdims=True), 1.0)
    lower = pooled / denom
    dot = functools.partial(jnp.dot, preferred_element_type=jnp.float32)
    pin = jnp.concatenate([inh_ref[...], lower], axis=1)
    hp = jax.nn.relu(dot(pin, wp1_ref[...]) + bp1_ref[...])
    hp = jax.nn.relu(dot(hp, wp2_ref[...]) + bp2_ref[...])
    prob = jax.nn.sigmoid(dot(hp, wprob_ref[...]) + bprob_ref[...])
    conf = jax.nn.sigmoid(dot(hp, wconf_ref[...]) + bconf_ref[...])
    hc = jax.nn.relu(dot(lower, wc1_ref[...]) + bc1_ref[...])
    logits_ref[...] = jax.nn.sigmoid(dot(hc, wc2_ref[...]) + bc2_ref[...])
    prob_ref[...] = prob
    conf_ref[...] = conf
    uu_ref[...] = (prob >= 0.5).astype(jnp.float32)
    lower_ref[...] = lower


def _head(hb, num_nodes, inh, wp1, bp1, wp2, bp2, wprob, bprob, wconf, bconf,
          wc1, bc1, wc2, bc2):
    full = lambda s: pl.BlockSpec(s, lambda: tuple(0 for _ in s))
    out_shapes = (jax.ShapeDtypeStruct((B, 1), jnp.float32),
                  jax.ShapeDtypeStruct((B, 1), jnp.float32),
                  jax.ShapeDtypeStruct((B, 1), jnp.float32),
                  jax.ShapeDtypeStruct((B, 1), jnp.float32),
                  jax.ShapeDtypeStruct((B, H), jnp.float32))
    return pl.pallas_call(
        _head_body,
        in_specs=[full((B, MAXN, H)), full((B, 1)), full((B, 8)),
                  full((8 + H, PHID)), full((1, PHID)),
                  full((PHID, PHID)), full((1, PHID)),
                  full((PHID, 1)), full((1, 1)),
                  full((PHID, 1)), full((1, 1)),
                  full((H, H // 2)), full((1, H // 2)),
                  full((H // 2, 1)), full((1, 1))],
        out_specs=(full((B, 1)), full((B, 1)), full((B, 1)), full((B, 1)),
                   full((B, H))),
        out_shape=out_shapes,
    )(hb, num_nodes.reshape(B, 1), inh, wp1, bp1.reshape(1, PHID),
      wp2, bp2.reshape(1, PHID), wprob, bprob.reshape(1, 1),
      wconf, bconf.reshape(1, 1), wc1, bc1.reshape(1, H // 2),
      wc2, bc2.reshape(1, 1))


def _sc_body(ht_hbm, flat_hbm, dst_hbm, zeros_hbm, out_hbm,
             idx_v, dst_v, rows_v, acc_sh, sem, isem):
    c = lax.axis_index("c")
    s = lax.axis_index("s")
    wid = s * 2 + c
    pltpu.sync_copy(zeros_hbm, acc_sh.at[pl.ds(s * RPS, RPS)])
    pltpu.sync_copy(flat_hbm.at[wid, pl.ds(0, SEG)], idx_v.at[0])
    pltpu.sync_copy(dst_hbm.at[wid, pl.ds(0, SEG)], dst_v.at[0])
    plsc.subcore_barrier()
    for b in range(NBUF):
        pltpu.async_copy(ht_hbm.at[idx_v.at[0, b]], rows_v.at[b], sem)

    for seg in range(NSEG):
        sl = seg % 2
        nx = (seg + 1) % 2
        if seg + 1 < NSEG:
            off = (seg + 1) * SEG
            pltpu.async_copy(flat_hbm.at[wid, pl.ds(off, SEG)],
                             idx_v.at[nx], isem)
            pltpu.async_copy(dst_hbm.at[wid, pl.ds(off, SEG)],
                             dst_v.at[nx], isem)

        def body(j0, carry):
            for b in range(NBUF):
                j = j0 * NBUF + b
                pltpu.make_async_copy(ht_hbm.at[idx_v.at[sl, j]],
                                      rows_v.at[b], sem).wait()
                pltpu.sync_copy(rows_v.at[b], acc_sh.at[dst_v.at[sl, j]],
                                add=True)
                pltpu.async_copy(ht_hbm.at[idx_v.at[sl, j + NBUF]],
                                 rows_v.at[b], sem)
            return carry

        lax.fori_loop(0, (SEG - NBUF) // NBUF, body, 0)

        if seg + 1 < NSEG:
            pltpu.make_async_copy(flat_hbm.at[wid, pl.ds(0, SEG)],
                                  idx_v.at[nx], isem).wait()
            pltpu.make_async_copy(dst_hbm.at[wid, pl.ds(0, SEG)],
                                  dst_v.at[nx], isem).wait()
        for b in range(NBUF):
            j = SEG - NBUF + b
            pltpu.make_async_copy(ht_hbm.at[idx_v.at[sl, j]],
                                  rows_v.at[b], sem).wait()
            pltpu.sync_copy(rows_v.at[b], acc_sh.at[dst_v.at[sl, j]],
                            add=True)
            if seg + 1 < NSEG:
                pltpu.async_copy(ht_hbm.at[idx_v.at[nx, b]],
                                 rows_v.at[b], sem)

    plsc.subcore_barrier()
    pltpu.sync_copy(acc_sh.at[pl.ds(s * RPS, RPS)],
                    out_hbm.at[c, pl.ds(s * RPS, RPS)])


def _sc_aggregate(ht, flat3, dst3, zeros):
    mesh = plsc.VectorSubcoreMesh(core_axis_name="c", subcore_axis_name="s")
    kern = pl.kernel(
        _sc_body,
        out_type=jax.ShapeDtypeStruct((2, N_PAD, H), jnp.float32),
        mesh=mesh,
        scratch_types=[
            pltpu.VMEM((2, SEG, CHUNK), jnp.int32),
            pltpu.VMEM((2, SEG, CHUNK), jnp.int32),
            pltpu.VMEM((NBUF, CHUNK, H), jnp.float32),
            pltpu.VMEM_SHARED((N_PAD, H), jnp.float32),
            pltpu.SemaphoreType.DMA,
            pltpu.SemaphoreType.DMA,
        ],
    )
    return kern(ht, flat3, dst3, zeros)


def kernel(node_features, edge_index, edge_type, num_nodes,
           inheritance_features, W_in, b_in, W_msg, b_msg, Wz, Uz, bz,
           Wr, Ur, br, Wn, Un, bn, Wp1, bp1, Wp2, bp2, Wprob, bprob,
           Wconf, bconf, Wc1, bc1, Wc2, bc2):
    x = node_features.reshape(N, F)
    src = edge_index[0]
    dst = edge_index[1]
    flat = edge_type * N + src
    epw = E // NWORK
    ppw = (E_PAD - E) // NWORK
    pad_flat = jnp.zeros((NWORK, ppw), jnp.int32)
    pad_dst = jnp.broadcast_to(
        N + (jnp.arange(ppw, dtype=jnp.int32) % (N_PAD - N)), (NWORK, ppw))
    flat3 = jnp.concatenate(
        [flat.reshape(NWORK, epw), pad_flat], axis=1).reshape(
            NWORK, NCHUNK, CHUNK)
    dst3 = jnp.concatenate(
        [dst.reshape(NWORK, epw), pad_dst], axis=1).reshape(
            NWORK, NCHUNK, CHUNK)
    zeros = jnp.zeros((RPS, H), jnp.float32)

    h = _encode(x, W_in, b_in)
    for l in range(L):
        ht = _msgs(h, W_msg[l], b_msg[l])
        partials = _sc_aggregate(ht, flat3, dst3, zeros)
        h = _gru(partials, h, Wz[l], Uz[l], bz[l], Wr[l], Ur[l], br[l],
                 Wn[l], Un[l], bn[l])

    logits, prob, conf, uu, lower = _head(
        h.reshape(B, MAXN, H), num_nodes, inheritance_features,
        Wp1, bp1, Wp2, bp2, Wprob, bprob, Wconf, bconf, Wc1, bc1, Wc2, bc2)
    return logits, prob, conf, uu, lower
